# Initial kernel scaffold; baseline (speedup 1.0000x reference)
import jax
import jax.numpy as jnp
from jax.experimental import pallas as pl


def kernel(features, rois):
    raise NotImplementedError("write your pallas kernel here")


# speedup vs baseline: 5.6706x; 5.6706x over previous
import functools

import jax
import jax.numpy as jnp
from jax import lax
from jax.experimental import pallas as pl
from jax.experimental.pallas import tpu as pltpu
from jax.experimental.pallas import tpu_sc as plsc

_B, _C, _H, _W = 2, 256, 50, 50
_N = 1000
_NPAD = 1024
_PH = _PW = 7
_NBINS = _PH * _PW
_SCALE = 0.0625
_NS = 16
_CHUNK = 16
_NCHUNK = _NPAD // _CHUNK
_CB = _CHUNK * _NBINS
_NGRP = _CB // 16
_NROWS = _B * _H * _W
_CG = 16
_NCG = _C // _CG


def _prep_body(rois_ref, idx_ref, w_ref):
    f = lax.broadcasted_iota(jnp.int32, (_NPAD, _NBINS * _NS), 1)
    bin_ = f // _NS
    s = f % _NS
    py = (bin_ // _PW).astype(jnp.float32)
    px = (bin_ % _PW).astype(jnp.float32)
    sy = ((s // 8) % 2).astype(jnp.float32)
    sx = ((s // 4) % 2).astype(jnp.float32)
    cy = (s // 2) % 2
    cx = s % 2

    rois = rois_ref[...]
    b = jnp.clip(rois[:, 0].astype(jnp.int32), 0, _B - 1)[:, None]
    x1 = (rois[:, 1] * _SCALE)[:, None]
    y1 = (rois[:, 2] * _SCALE)[:, None]
    x2 = (rois[:, 3] * _SCALE)[:, None]
    y2 = (rois[:, 4] * _SCALE)[:, None]
    roi_w = jnp.maximum(x2 - x1, 1.0)
    roi_h = jnp.maximum(y2 - y1, 1.0)
    bin_w = roi_w * (1.0 / _PW)
    bin_h = roi_h * (1.0 / _PH)

    ys = y1 + py * bin_h + (sy + 0.5) * bin_h * 0.5
    xs = x1 + px * bin_w + (sx + 0.5) * bin_w * 0.5
    valid = (ys > -1.0) & (ys < float(_H)) & (xs > -1.0) & (xs < float(_W))
    ycl = jnp.clip(ys, 0.0, float(_H - 1))
    xcl = jnp.clip(xs, 0.0, float(_W - 1))
    ylf = jnp.floor(ycl)
    xlf = jnp.floor(xcl)
    y_low = ylf.astype(jnp.int32)
    x_low = xlf.astype(jnp.int32)
    ly = ycl - ylf
    lx = xcl - xlf
    wy = jnp.where(cy == 1, ly, 1.0 - ly)
    wx = jnp.where(cx == 1, lx, 1.0 - lx)
    y_cor = jnp.where(cy == 1, jnp.minimum(y_low + 1, _H - 1), y_low)
    x_cor = jnp.where(cx == 1, jnp.minimum(x_low + 1, _W - 1), x_low)

    w = jnp.where(valid, wy * wx * 0.25, 0.0)
    idx = (b * (_H * _W) + y_cor * _W + x_cor) * _CG
    idx_ref[...] = idx
    w_ref[...] = w


def _sc_body(table_hbm, idx_hbm, w_hbm, out_hbm, table_v, idx_v, w_v, buf):
    core = lax.axis_index("c")
    cg = lax.axis_index("s")
    pltpu.sync_copy(table_hbm.at[cg], table_v)

    def chunk_body(ci, carry):
        chunk = core * (_NCHUNK // 2) + ci
        pltpu.sync_copy(idx_hbm.at[chunk], idx_v)
        pltpu.sync_copy(w_hbm.at[chunk], w_v)

        def group_body(j, carry2):
            accs = [jnp.zeros((16,), jnp.float32) for _ in range(_CG)]
            for si in range(_NS):
                iv = idx_v[j, si]
                wv = w_v[j, si]
                for ch in range(_CG):
                    g = plsc.load_gather(table_v, [iv + ch])
                    accs[ch] = accs[ch] + wv * g
            for ch in range(_CG):
                buf[ch, pl.ds(j * 16, 16)] = accs[ch]
            return carry2

        lax.fori_loop(0, _NGRP, group_body, 0, unroll=False)
        pltpu.sync_copy(buf, out_hbm.at[chunk, cg])
        return carry

    lax.fori_loop(0, _NCHUNK // 2, chunk_body, 0, unroll=False)


@jax.jit
def kernel(features, rois):
    table = (
        features.transpose(1, 0, 2, 3)
        .reshape(_NCG, _CG, _NROWS)
        .transpose(0, 2, 1)
        .reshape(_NCG, _NROWS * _CG)
    )
    rois_p = jnp.zeros((_NPAD, 5), jnp.float32).at[:_N].set(rois)

    idx_n, w_n = pl.pallas_call(
        _prep_body,
        out_shape=(
            jax.ShapeDtypeStruct((_NPAD, _NBINS * _NS), jnp.int32),
            jax.ShapeDtypeStruct((_NPAD, _NBINS * _NS), jnp.float32),
        ),
    )(rois_p)

    def _re(a):
        a = a.reshape(_NCHUNK, _CB, _NS)
        a = a.reshape(_NCHUNK, _NGRP, 16, _NS)
        return a.transpose(0, 1, 3, 2)

    idx_sc = _re(idx_n)
    w_sc = _re(w_n)

    mesh = plsc.VectorSubcoreMesh(core_axis_name="c", subcore_axis_name="s")
    out = pl.kernel(
        _sc_body,
        out_type=jax.ShapeDtypeStruct((_NCHUNK, _NCG, _CG, _CB), jnp.float32),
        mesh=mesh,
        scratch_types=[
            pltpu.VMEM((_NROWS * _CG,), jnp.float32),
            pltpu.VMEM((_NGRP, _NS, 16), jnp.int32),
            pltpu.VMEM((_NGRP, _NS, 16), jnp.float32),
            pltpu.VMEM((_CG, _CB), jnp.float32),
        ],
        compiler_params=pltpu.CompilerParams(
            needs_layout_passes=False, use_tc_tiling_on_sc=False
        ),
    )(table, idx_sc, w_sc)

    out = out.reshape(_NCHUNK, _NCG, _CG, _CHUNK, _NBINS)
    out = out.transpose(0, 3, 1, 2, 4).reshape(_NPAD, _C, _NBINS)
    return out[:_N].reshape(_N, _C, _PH, _PW)
